# Initial kernel scaffold; baseline (speedup 1.0000x reference)
import jax
import jax.numpy as jnp
from jax.experimental import pallas as pl


def kernel(x, edge_index, deg_hist, pre_W1, pre_b1, post_W1, post_b1, lin_W1, lin_b1, pre_W2, pre_b2, post_W2, post_b2, lin_W2, lin_b2):
    raise NotImplementedError("write your pallas kernel here")


# speedup vs baseline: 1.1810x; 1.1810x over previous
import functools

import jax
import jax.numpy as jnp
from jax.experimental import pallas as pl
from jax.experimental.pallas import tpu as pltpu

N_NODES = 10000
F = 128
EPS = 1e-5
BLK = 400


def _pre_body(x_ref, wt_ref, wb_ref, pb_ref, a_ref, b_ref):
    x = x_ref[...]
    a_ref[...] = (
        jnp.dot(x, wt_ref[...], preferred_element_type=jnp.float32) + pb_ref[...]
    )
    b_ref[...] = jnp.dot(x, wb_ref[...], preferred_element_type=jnp.float32)


def _pre(x, pre_W, pre_b):
    wt, wb = pre_W[:F], pre_W[F:]
    a, b = pl.pallas_call(
        _pre_body,
        grid=(N_NODES // BLK,),
        in_specs=[
            pl.BlockSpec((BLK, F), lambda i: (i, 0)),
            pl.BlockSpec((F, F), lambda i: (0, 0)),
            pl.BlockSpec((F, F), lambda i: (0, 0)),
            pl.BlockSpec((1, F), lambda i: (0, 0)),
        ],
        out_specs=[
            pl.BlockSpec((BLK, F), lambda i: (i, 0)),
            pl.BlockSpec((BLK, F), lambda i: (i, 0)),
        ],
        out_shape=[jax.ShapeDtypeStruct((N_NODES, F), jnp.float32)] * 2,
    )(x, wt, wb, pre_b.reshape(1, F))
    return a, b


def _post_body(apply_elu, x_ref, a_ref, cnt_ref, s_ref, s2_ref, mn_ref, mx_ref,
               px_ref, p1_ref, p2_ref, p3_ref, pb_ref, lw_ref, lb_ref,
               al_ref, o_ref):
    a = a_ref[...]
    cnt = cnt_ref[...]
    d = jnp.maximum(cnt, 1.0)
    has = cnt > 0.0
    s = cnt * a + s_ref[...]
    mean = s / d
    mean2 = (cnt * a * a + 2.0 * a * s_ref[...] + s2_ref[...]) / d
    var = jnp.maximum(mean2 - mean * mean, 0.0)
    std = jnp.sqrt(var + EPS)
    mn = jnp.where(has, a + mn_ref[...], 0.0)
    mx = jnp.where(has, a + mx_ref[...], 0.0)
    agg = jnp.concatenate([mean, mn, mx, std], axis=-1)
    avg_log = al_ref[0, 0]
    logd = jnp.log(d + 1.0)
    amp = logd / avg_log
    att = avg_log / logd
    t1 = jnp.dot(agg, p1_ref[...], preferred_element_type=jnp.float32)
    t2 = jnp.dot(agg, p2_ref[...], preferred_element_type=jnp.float32)
    t3 = jnp.dot(agg, p3_ref[...], preferred_element_type=jnp.float32)
    out = (
        jnp.dot(x_ref[...], px_ref[...], preferred_element_type=jnp.float32)
        + t1 + amp * t2 + att * t3 + pb_ref[...]
    )
    out = jnp.dot(out, lw_ref[...], preferred_element_type=jnp.float32) + lb_ref[...]
    if apply_elu:
        out = jnp.where(out > 0.0, out, jnp.exp(jnp.minimum(out, 0.0)) - 1.0)
    o_ref[...] = out


def _post(x, a, cnt, S, S2, MN, MX, post_W, post_b, lin_W, lin_b, avg_log,
          apply_elu):
    px, p1, p2, p3 = (post_W[:F], post_W[F:F + 4 * F],
                      post_W[F + 4 * F:F + 8 * F], post_W[F + 8 * F:])
    full = lambda shp: pl.BlockSpec(shp, lambda i: (0, 0))
    blk = pl.BlockSpec((BLK, F), lambda i: (i, 0))
    return pl.pallas_call(
        functools.partial(_post_body, apply_elu),
        grid=(N_NODES // BLK,),
        in_specs=[
            blk, blk,
            pl.BlockSpec((BLK, 1), lambda i: (i, 0)),
            blk, blk, blk, blk,
            full((F, F)), full((4 * F, F)), full((4 * F, F)), full((4 * F, F)),
            full((1, F)), full((F, F)), full((1, F)), full((1, 1)),
        ],
        out_specs=blk,
        out_shape=jax.ShapeDtypeStruct((N_NODES, F), jnp.float32),
    )(x, a, cnt.reshape(-1, 1), S, S2, MN, MX, px, p1, p2, p3,
      post_b.reshape(1, F), lin_W, lin_b.reshape(1, F), avg_log.reshape(1, 1))


def _segment(b, src, dst):
    bs = b[src]
    ones = jnp.ones((dst.shape[0],), jnp.float32)
    cnt = jax.ops.segment_sum(ones, dst, num_segments=N_NODES)
    S = jax.ops.segment_sum(bs, dst, num_segments=N_NODES)
    S2 = jax.ops.segment_sum(bs * bs, dst, num_segments=N_NODES)
    MN = jax.ops.segment_min(bs, dst, num_segments=N_NODES)
    MX = jax.ops.segment_max(bs, dst, num_segments=N_NODES)
    return cnt, S, S2, MN, MX


def _layer(x, src, dst, avg_log, pre_W, pre_b, post_W, post_b, lin_W, lin_b,
           apply_elu):
    a, b = _pre(x, pre_W, pre_b)
    cnt, S, S2, MN, MX = _segment(b, src, dst)
    return _post(x, a, cnt, S, S2, MN, MX, post_W, post_b, lin_W, lin_b,
                 avg_log, apply_elu)


def kernel(x, edge_index, deg_hist, pre_W1, pre_b1, post_W1, post_b1, lin_W1,
           lin_b1, pre_W2, pre_b2, post_W2, post_b2, lin_W2, lin_b2):
    src, dst = edge_index[0], edge_index[1]
    bins = jnp.arange(deg_hist.shape[0], dtype=jnp.float32)
    hist = deg_hist.astype(jnp.float32)
    avg_log = jnp.sum(jnp.log(bins + 1.0) * hist) / jnp.sum(hist)
    h = _layer(x, src, dst, avg_log, pre_W1, pre_b1, post_W1, post_b1,
               lin_W1, lin_b1, True)
    return _layer(h, src, dst, avg_log, pre_W2, pre_b2, post_W2, post_b2,
                  lin_W2, lin_b2, False)


# speedup vs baseline: 4.8384x; 4.0970x over previous
import functools

import jax
import jax.numpy as jnp
from jax import lax
from jax.experimental import pallas as pl
from jax.experimental.pallas import tpu as pltpu
from jax.experimental.pallas import tpu_sc as plsc

N_NODES = 10000
E_EDGES = 320000
F = 128
EPS = 1e-5
BLK = 400

NC = 2
NS = 16
NW = NC * NS
NR = 64
NPR = 157
NPAD = NR * NPR
CAP = 8192
LISTSZ = CAP + 144
CHUNK = 4000
NCHUNK = E_EDGES // CHUNK
BATCH = 128
SUB = BATCH // 16
RACC = (NPR + 1) * F
ROUT = NPR * F
RCNT = (NPR + 1) * 16
BIG = 3.0e38

_MESH = plsc.VectorSubcoreMesh(core_axis_name="c", subcore_axis_name="s")
_SC_PARAMS = pltpu.CompilerParams(needs_layout_passes=False)


def _wid():
    return lax.axis_index("s") * NC + lax.axis_index("c")


def _partition_body(src_hbm, dst_hbm, osrc_hbm, odst_hbm, cnts_hbm,
                    src_v, dst_v, osrc_l0, odst_l0, osrc_l1, odst_l1, cnt_v):
    wid = _wid()
    base_lo = wid * (2 * NPR)

    def chunk_body(c, carry):
        pltpu.sync_copy(src_hbm.at[pl.ds(c * CHUNK, CHUNK)], src_v)
        pltpu.sync_copy(dst_hbm.at[pl.ds(c * CHUNK, CHUNK)], dst_v)

        def scan_body(i, carry):
            off0, off1 = carry
            s16 = src_v[pl.ds(i * 16, 16)]
            d16 = dst_v[pl.ds(i * 16, 16)]
            dloc = d16 - base_lo
            m0 = (dloc >= 0) & (dloc < NPR)
            m1 = (dloc >= NPR) & (dloc < 2 * NPR)
            cs0 = plsc.cumsum(m0.astype(jnp.int32))
            cs1 = plsc.cumsum(m1.astype(jnp.int32))
            p0 = jnp.minimum(off0, CAP) + cs0 - 1
            p1 = jnp.minimum(off1, CAP) + cs1 - 1
            plsc.store_scatter(osrc_l0, [p0], s16, mask=m0)
            plsc.store_scatter(odst_l0, [p0], dloc, mask=m0)
            plsc.store_scatter(osrc_l1, [p1], s16, mask=m1)
            plsc.store_scatter(odst_l1, [p1], dloc - NPR, mask=m1)
            return (off0 + cs0[15], off1 + cs1[15])

        return lax.fori_loop(0, CHUNK // 16, scan_body, carry)

    off0, off1 = lax.fori_loop(0, NCHUNK, chunk_body,
                               (jnp.int32(0), jnp.int32(0)))
    zeros = jnp.zeros((16,), jnp.int32)
    trash = jnp.full((16,), NPR, jnp.int32)
    for ol, dl_, off in ((osrc_l0, odst_l0, off0), (osrc_l1, odst_l1, off1)):
        offc = jnp.minimum(off, CAP)
        for p in range(8):
            ol[pl.ds(offc + 16 * p, 16)] = zeros
            dl_[pl.ds(offc + 16 * p, 16)] = trash
    cnt_v[...] = jnp.full((16,), jnp.minimum(off0, CAP), jnp.int32)
    pltpu.sync_copy(cnt_v, cnts_hbm.at[2 * wid])
    cnt_v[...] = jnp.full((16,), jnp.minimum(off1, CAP), jnp.int32)
    pltpu.sync_copy(cnt_v, cnts_hbm.at[2 * wid + 1])
    pltpu.sync_copy(osrc_l0, osrc_hbm.at[2 * wid])
    pltpu.sync_copy(odst_l0, odst_hbm.at[2 * wid])
    pltpu.sync_copy(osrc_l1, osrc_hbm.at[2 * wid + 1])
    pltpu.sync_copy(odst_l1, odst_hbm.at[2 * wid + 1])


_partition = functools.partial(
    pl.kernel, _partition_body, mesh=_MESH,
    compiler_params=_SC_PARAMS,
    out_type=[
        jax.ShapeDtypeStruct((NR, LISTSZ), jnp.int32),
        jax.ShapeDtypeStruct((NR, LISTSZ), jnp.int32),
        jax.ShapeDtypeStruct((NR, 16), jnp.int32),
    ],
    scratch_types=[
        pltpu.VMEM((CHUNK,), jnp.int32),
        pltpu.VMEM((CHUNK,), jnp.int32),
        pltpu.VMEM((LISTSZ,), jnp.int32),
        pltpu.VMEM((LISTSZ,), jnp.int32),
        pltpu.VMEM((LISTSZ,), jnp.int32),
        pltpu.VMEM((LISTSZ,), jnp.int32),
        pltpu.VMEM((16,), jnp.int32),
    ],
)()


def _segment_body(with_cnt, b, osrc_hbm, odst_hbm, cnts_hbm, *refs):
    if with_cnt:
        (s_out, s2_out, mn_out, mx_out, cnt_out,
         osrc_v, odst_v, stag, acc_s, acc_s2, acc_mn, acc_mx, acc_cnt,
         cnt_v, sem) = refs
    else:
        (s_out, s2_out, mn_out, mx_out,
         osrc_v, odst_v, stag, acc_s, acc_s2, acc_mn, acc_mx,
         cnt_v, sem) = refs
    wid = _wid()
    zf = jnp.zeros((16,), jnp.float32)
    big = jnp.full((16,), BIG, jnp.float32)
    ones = jnp.ones((16,), jnp.float32)
    iot = lax.iota(jnp.int32, 16)

    for vr in (0, 1):
        vw = 2 * wid + vr
        pltpu.sync_copy(osrc_hbm.at[vw], osrc_v)
        pltpu.sync_copy(odst_hbm.at[vw], odst_v)
        pltpu.sync_copy(cnts_hbm.at[vw], cnt_v)
        cw = cnt_v[...][0]
        nb = (cw + BATCH - 1) // BATCH

        def init_body(i, _):
            acc_s[pl.ds(i * 16, 16)] = zf
            acc_s2[pl.ds(i * 16, 16)] = zf
            acc_mn[pl.ds(i * 16, 16)] = big
            acc_mx[pl.ds(i * 16, 16)] = -big
            return 0

        lax.fori_loop(0, RACC // 16, init_body, 0)
        if with_cnt:
            def cinit_body(i, _):
                acc_cnt[pl.ds(i * 16, 16)] = zf
                return 0

            lax.fori_loop(0, RCNT // 16, cinit_body, 0)

        def batch_body(j, _):
            jb = pl.multiple_of(j * BATCH, BATCH)
            pltpu.async_copy(b.at[osrc_v.at[pl.ds(jb, BATCH)]], stag, sem).wait()

            def sub_body(s, _):
                o16 = odst_v[pl.ds(j * BATCH + s * 16, 16)]
                for e in range(16):
                    dl = o16[e]
                    rb = dl * F
                    rsp = jnp.full((16,), s * 16 + e, jnp.int32)
                    for k in range(F // 16):
                        v = plsc.load_gather(stag, [rsp, iot + k * 16])
                        plsc.addupdate(acc_s.at[pl.ds(rb + k * 16, 16)], v)
                        plsc.addupdate(acc_s2.at[pl.ds(rb + k * 16, 16)], v * v)
                        cm = acc_mn[pl.ds(rb + k * 16, 16)]
                        acc_mn[pl.ds(rb + k * 16, 16)] = jnp.minimum(cm, v)
                        cx = acc_mx[pl.ds(rb + k * 16, 16)]
                        acc_mx[pl.ds(rb + k * 16, 16)] = jnp.maximum(cx, v)
                    if with_cnt:
                        plsc.addupdate(acc_cnt.at[pl.ds(dl * 16, 16)], ones)
                return 0

            lax.fori_loop(0, SUB, sub_body, 0)
            return 0

        lax.fori_loop(0, nb, batch_body, 0)
        base = pl.multiple_of(vw * ROUT, 128)
        pltpu.sync_copy(acc_s.at[pl.ds(0, ROUT)], s_out.at[pl.ds(base, ROUT)])
        pltpu.sync_copy(acc_s2.at[pl.ds(0, ROUT)], s2_out.at[pl.ds(base, ROUT)])
        pltpu.sync_copy(acc_mn.at[pl.ds(0, ROUT)], mn_out.at[pl.ds(base, ROUT)])
        pltpu.sync_copy(acc_mx.at[pl.ds(0, ROUT)], mx_out.at[pl.ds(base, ROUT)])
        if with_cnt:
            pltpu.sync_copy(acc_cnt, cnt_out.at[vw])


def _make_segment(with_cnt):
    outs = [jax.ShapeDtypeStruct((NR * ROUT,), jnp.float32)] * 4
    scr = [
        pltpu.VMEM((LISTSZ,), jnp.int32),
        pltpu.VMEM((LISTSZ,), jnp.int32),
        pltpu.VMEM((BATCH, F), jnp.float32),
        pltpu.VMEM((RACC,), jnp.float32),
        pltpu.VMEM((RACC,), jnp.float32),
        pltpu.VMEM((RACC,), jnp.float32),
        pltpu.VMEM((RACC,), jnp.float32),
    ]
    if with_cnt:
        outs = outs + [jax.ShapeDtypeStruct((NR, RCNT), jnp.float32)]
        scr = scr + [pltpu.VMEM((RCNT,), jnp.float32)]
    scr = scr + [pltpu.VMEM((16,), jnp.int32), pltpu.SemaphoreType.DMA]
    return functools.partial(
        pl.kernel, functools.partial(_segment_body, with_cnt),
        mesh=_MESH, compiler_params=_SC_PARAMS,
        out_type=outs, scratch_types=scr)()


_segment_cnt = _make_segment(True)
_segment_nocnt = _make_segment(False)


def _assemble(flat):
    return flat.reshape(NPAD, F)[:N_NODES]


def _pre_body(x_ref, wt_ref, wb_ref, pb_ref, a_ref, b_ref):
    x = x_ref[...]
    a_ref[...] = (
        jnp.dot(x, wt_ref[...], preferred_element_type=jnp.float32) + pb_ref[...]
    )
    b_ref[...] = jnp.dot(x, wb_ref[...], preferred_element_type=jnp.float32)


def _pre(x, pre_W, pre_b):
    wt, wb = pre_W[:F], pre_W[F:]
    return pl.pallas_call(
        _pre_body,
        grid=(N_NODES // BLK,),
        in_specs=[
            pl.BlockSpec((BLK, F), lambda i: (i, 0)),
            pl.BlockSpec((F, F), lambda i: (0, 0)),
            pl.BlockSpec((F, F), lambda i: (0, 0)),
            pl.BlockSpec((1, F), lambda i: (0, 0)),
        ],
        out_specs=[
            pl.BlockSpec((BLK, F), lambda i: (i, 0)),
            pl.BlockSpec((BLK, F), lambda i: (i, 0)),
        ],
        out_shape=[
            jax.ShapeDtypeStruct((N_NODES, F), jnp.float32),
            jax.ShapeDtypeStruct((N_NODES, F), jnp.float32),
        ],
    )(x, wt, wb, pre_b.reshape(1, F))


def _post_body(apply_elu, x_ref, a_ref, cnt_ref, s_ref, s2_ref, mn_ref, mx_ref,
               px_ref, p1_ref, p2_ref, p3_ref, pb_ref, lw_ref, lb_ref,
               al_ref, o_ref):
    a = a_ref[...]
    cnt = cnt_ref[...]
    d = jnp.maximum(cnt, 1.0)
    has = cnt > 0.0
    s = cnt * a + s_ref[...]
    mean = s / d
    mean2 = (cnt * a * a + 2.0 * a * s_ref[...] + s2_ref[...]) / d
    var = jnp.maximum(mean2 - mean * mean, 0.0)
    std = jnp.sqrt(var + EPS)
    mn = jnp.where(has, a + mn_ref[...], 0.0)
    mx = jnp.where(has, a + mx_ref[...], 0.0)
    agg = jnp.concatenate([mean, mn, mx, std], axis=-1)
    avg_log = al_ref[0, 0]
    logd = jnp.log(d + 1.0)
    amp = logd / avg_log
    att = avg_log / logd
    t1 = jnp.dot(agg, p1_ref[...], preferred_element_type=jnp.float32)
    t2 = jnp.dot(agg, p2_ref[...], preferred_element_type=jnp.float32)
    t3 = jnp.dot(agg, p3_ref[...], preferred_element_type=jnp.float32)
    out = (
        jnp.dot(x_ref[...], px_ref[...], preferred_element_type=jnp.float32)
        + t1 + amp * t2 + att * t3 + pb_ref[...]
    )
    out = jnp.dot(out, lw_ref[...], preferred_element_type=jnp.float32) + lb_ref[...]
    if apply_elu:
        out = jnp.where(out > 0.0, out, jnp.exp(jnp.minimum(out, 0.0)) - 1.0)
    o_ref[...] = out


def _post(x, a, cnt, S, S2, MN, MX, post_W, post_b, lin_W, lin_b, avg_log,
          apply_elu):
    px, p1, p2, p3 = (post_W[:F], post_W[F:F + 4 * F],
                      post_W[F + 4 * F:F + 8 * F], post_W[F + 8 * F:])
    full = lambda shp: pl.BlockSpec(shp, lambda i: (0, 0))
    blk = pl.BlockSpec((BLK, F), lambda i: (i, 0))
    return pl.pallas_call(
        functools.partial(_post_body, apply_elu),
        grid=(N_NODES // BLK,),
        in_specs=[
            blk, blk,
            pl.BlockSpec((BLK, 1), lambda i: (i, 0)),
            blk, blk, blk, blk,
            full((F, F)), full((4 * F, F)), full((4 * F, F)), full((4 * F, F)),
            full((1, F)), full((F, F)), full((1, F)), full((1, 1)),
        ],
        out_specs=blk,
        out_shape=jax.ShapeDtypeStruct((N_NODES, F), jnp.float32),
    )(x, a, cnt.reshape(-1, 1), S, S2, MN, MX, px, p1, p2, p3,
      post_b.reshape(1, F), lin_W, lin_b.reshape(1, F), avg_log.reshape(1, 1))


def kernel(x, edge_index, deg_hist, pre_W1, pre_b1, post_W1, post_b1, lin_W1,
           lin_b1, pre_W2, pre_b2, post_W2, post_b2, lin_W2, lin_b2):
    src, dst = edge_index[0], edge_index[1]
    bins = jnp.arange(deg_hist.shape[0], dtype=jnp.float32)
    hist = deg_hist.astype(jnp.float32)
    avg_log = jnp.sum(jnp.log(bins + 1.0) * hist) / jnp.sum(hist)

    osrc, odst, cnts = _partition(src, dst)

    a1, b1 = _pre(x, pre_W1, pre_b1)
    sS, sS2, sMN, sMX, sCNT = _segment_cnt(b1, osrc, odst, cnts)
    cnt = sCNT.reshape(NR, NPR + 1, 16)[:, :NPR, 0].reshape(NPAD)[:N_NODES]
    h = _post(x, a1, cnt, _assemble(sS), _assemble(sS2), _assemble(sMN),
              _assemble(sMX), post_W1, post_b1, lin_W1, lin_b1, avg_log, True)

    a2, b2 = _pre(h, pre_W2, pre_b2)
    tS, tS2, tMN, tMX = _segment_nocnt(b2, osrc, odst, cnts)
    return _post(h, a2, cnt, _assemble(tS), _assemble(tS2), _assemble(tMN),
                 _assemble(tMX), post_W2, post_b2, lin_W2, lin_b2, avg_log,
                 False)


# speedup vs baseline: 5.5408x; 1.1452x over previous
import functools

import jax
import jax.numpy as jnp
from jax import lax
from jax.experimental import pallas as pl
from jax.experimental.pallas import tpu as pltpu
from jax.experimental.pallas import tpu_sc as plsc

N_NODES = 10000
E_EDGES = 320000
F = 128
EPS = 1e-5
BLK = 400

NC = 2
NS = 16
NW = NC * NS
NR = 64
NPR = 157
NPAD = NR * NPR
CAP = 8192
LISTSZ = CAP + 144
CHUNK = 16000
NCHUNK = E_EDGES // CHUNK
BATCH = 128
SUB = BATCH // 16
RACC = (NPR + 1) * F
ROUT = NPR * F
RCNT = (NPR + 1) * 16
BIG = 3.0e38

_MESH = plsc.VectorSubcoreMesh(core_axis_name="c", subcore_axis_name="s")
_SC_PARAMS = pltpu.CompilerParams(needs_layout_passes=False)


def _wid():
    return lax.axis_index("s") * NC + lax.axis_index("c")


def _partition_body(src_hbm, dst_hbm, osrc_hbm, odst_hbm, cnts_hbm, deg_hbm,
                    src_v, dst_v, osrc_l0, odst_l0, osrc_l1, odst_l1, cnt_v,
                    acc_cnt):
    wid = _wid()
    base_lo = wid * (2 * NPR)

    def chunk_body(c, carry):
        pltpu.sync_copy(src_hbm.at[pl.ds(c * CHUNK, CHUNK)], src_v)
        pltpu.sync_copy(dst_hbm.at[pl.ds(c * CHUNK, CHUNK)], dst_v)

        def scan_body(i, carry):
            off0, off1 = carry
            s16 = src_v[pl.ds(i * 16, 16)]
            d16 = dst_v[pl.ds(i * 16, 16)]
            dloc = d16 - base_lo
            m0 = (dloc >= 0) & (dloc < NPR)
            m1 = (dloc >= NPR) & (dloc < 2 * NPR)
            cs0 = plsc.cumsum(m0.astype(jnp.int32))
            cs1 = plsc.cumsum(m1.astype(jnp.int32))
            p0 = jnp.minimum(off0, CAP) + cs0 - 1
            p1 = jnp.minimum(off1, CAP) + cs1 - 1
            plsc.store_scatter(osrc_l0, [p0], s16, mask=m0)
            plsc.store_scatter(odst_l0, [p0], dloc, mask=m0)
            plsc.store_scatter(osrc_l1, [p1], s16, mask=m1)
            plsc.store_scatter(odst_l1, [p1], dloc - NPR, mask=m1)
            return (off0 + cs0[15], off1 + cs1[15])

        return lax.fori_loop(0, CHUNK // 16, scan_body, carry)

    off0, off1 = lax.fori_loop(0, NCHUNK, chunk_body,
                               (jnp.int32(0), jnp.int32(0)))
    zeros = jnp.zeros((16,), jnp.int32)
    trash = jnp.full((16,), NPR, jnp.int32)
    for ol, dl_, off in ((osrc_l0, odst_l0, off0), (osrc_l1, odst_l1, off1)):
        offc = jnp.minimum(off, CAP)
        for p in range(8):
            ol[pl.ds(offc + 16 * p, 16)] = zeros
            dl_[pl.ds(offc + 16 * p, 16)] = trash
    cnt_v[...] = jnp.full((16,), jnp.minimum(off0, CAP), jnp.int32)
    pltpu.sync_copy(cnt_v, cnts_hbm.at[2 * wid])
    cnt_v[...] = jnp.full((16,), jnp.minimum(off1, CAP), jnp.int32)
    pltpu.sync_copy(cnt_v, cnts_hbm.at[2 * wid + 1])
    pltpu.sync_copy(osrc_l0, osrc_hbm.at[2 * wid])
    pltpu.sync_copy(odst_l0, odst_hbm.at[2 * wid])
    pltpu.sync_copy(osrc_l1, osrc_hbm.at[2 * wid + 1])
    pltpu.sync_copy(odst_l1, odst_hbm.at[2 * wid + 1])
    onesf = jnp.ones((16,), jnp.float32)
    zerosf = jnp.zeros((16,), jnp.float32)
    for vr, (dl_, off) in enumerate(((odst_l0, off0), (odst_l1, off1))):
        def czero(i, _):
            acc_cnt[pl.ds(i * 16, 16)] = zerosf
            return 0

        lax.fori_loop(0, RCNT // 16, czero, 0)
        nbc = (jnp.minimum(off, CAP) + 15) // 16

        def cbody(i, _, dl_=dl_):
            o16 = dl_[pl.ds(i * 16, 16)]
            for e in range(16):
                d = o16[e]
                plsc.addupdate(acc_cnt.at[pl.ds(d * 16, 16)], onesf)
            return 0

        lax.fori_loop(0, nbc, cbody, 0)
        pltpu.sync_copy(acc_cnt, deg_hbm.at[2 * wid + vr])


_partition = functools.partial(
    pl.kernel, _partition_body, mesh=_MESH,
    compiler_params=_SC_PARAMS,
    out_type=[
        jax.ShapeDtypeStruct((NR, LISTSZ), jnp.int32),
        jax.ShapeDtypeStruct((NR, LISTSZ), jnp.int32),
        jax.ShapeDtypeStruct((NR, 16), jnp.int32),
        jax.ShapeDtypeStruct((NR, RCNT), jnp.float32),
    ],
    scratch_types=[
        pltpu.VMEM((CHUNK,), jnp.int32),
        pltpu.VMEM((CHUNK,), jnp.int32),
        pltpu.VMEM((LISTSZ,), jnp.int32),
        pltpu.VMEM((LISTSZ,), jnp.int32),
        pltpu.VMEM((LISTSZ,), jnp.int32),
        pltpu.VMEM((LISTSZ,), jnp.int32),
        pltpu.VMEM((16,), jnp.int32),
        pltpu.VMEM((RCNT,), jnp.float32),
    ],
)()


def _segment_body(b, osrc_hbm, odst_hbm, cnts_hbm,
                  s_out, s2_out, mn_out, mx_out,
                  osrc_v, odst_v, stag0, stag1, acc_s, acc_s2, acc_mn, acc_mx,
                  cnt_v, sem0, sem1):
    wid = _wid()
    zf = jnp.zeros((16,), jnp.float32)
    big = jnp.full((16,), BIG, jnp.float32)

    for vr in (0, 1):
        vw = 2 * wid + vr
        pltpu.sync_copy(osrc_hbm.at[vw], osrc_v)
        pltpu.sync_copy(odst_hbm.at[vw], odst_v)
        pltpu.sync_copy(cnts_hbm.at[vw], cnt_v)
        cw = cnt_v[...][0]
        nb = (cw + BATCH - 1) // BATCH

        def init_body(i, _):
            acc_s[pl.ds(i * 16, 16)] = zf
            acc_s2[pl.ds(i * 16, 16)] = zf
            acc_mn[pl.ds(i * 16, 16)] = big
            acc_mx[pl.ds(i * 16, 16)] = -big
            return 0

        lax.fori_loop(0, RACC // 16, init_body, 0)

        def issue(j, stagx, semx):
            jb = pl.multiple_of(j * BATCH, BATCH)
            pltpu.async_copy(b.at[osrc_v.at[pl.ds(jb, BATCH)]], stagx, semx)

        def waitb(stagx, semx):
            pltpu.make_async_copy(
                b.at[osrc_v.at[pl.ds(0, BATCH)]], stagx, semx).wait()

        def accum(j, stagx):
            def sub_body(s, _):
                o16 = odst_v[pl.ds(j * BATCH + s * 16, 16)]
                for e in range(16):
                    dl = o16[e]
                    rb = dl * F
                    r = s * 16 + e
                    for k in range(F // 16):
                        v = stagx[r, pl.ds(k * 16, 16)]
                        plsc.addupdate(acc_s.at[pl.ds(rb + k * 16, 16)], v)
                        plsc.addupdate(acc_s2.at[pl.ds(rb + k * 16, 16)], v * v)
                        cm = acc_mn[pl.ds(rb + k * 16, 16)]
                        acc_mn[pl.ds(rb + k * 16, 16)] = jnp.minimum(cm, v)
                        cx = acc_mx[pl.ds(rb + k * 16, 16)]
                        acc_mx[pl.ds(rb + k * 16, 16)] = jnp.maximum(cx, v)
                return 0

            lax.fori_loop(0, SUB, sub_body, 0)

        @pl.when(nb > 0)
        def _():
            issue(0, stag0, sem0)

        def pair_body(p, _):
            j0 = 2 * p
            j1 = j0 + 1

            @pl.when(j1 < nb)
            def _():
                issue(j1, stag1, sem1)

            waitb(stag0, sem0)
            accum(j0, stag0)

            @pl.when(j1 < nb)
            def _():
                @pl.when(j1 + 1 < nb)
                def _():
                    issue(j1 + 1, stag0, sem0)

                waitb(stag1, sem1)
                accum(j1, stag1)

            return 0

        lax.fori_loop(0, (nb + 1) // 2, pair_body, 0)
        base = pl.multiple_of(vw * ROUT, 128)
        pltpu.sync_copy(acc_s.at[pl.ds(0, ROUT)], s_out.at[pl.ds(base, ROUT)])
        pltpu.sync_copy(acc_s2.at[pl.ds(0, ROUT)], s2_out.at[pl.ds(base, ROUT)])
        pltpu.sync_copy(acc_mn.at[pl.ds(0, ROUT)], mn_out.at[pl.ds(base, ROUT)])
        pltpu.sync_copy(acc_mx.at[pl.ds(0, ROUT)], mx_out.at[pl.ds(base, ROUT)])


_segment = functools.partial(
    pl.kernel, _segment_body,
    mesh=_MESH, compiler_params=_SC_PARAMS,
    out_type=[jax.ShapeDtypeStruct((NR * ROUT,), jnp.float32)] * 4,
    scratch_types=[
        pltpu.VMEM((LISTSZ,), jnp.int32),
        pltpu.VMEM((LISTSZ,), jnp.int32),
        pltpu.VMEM((BATCH, F), jnp.float32),
        pltpu.VMEM((BATCH, F), jnp.float32),
        pltpu.VMEM((RACC,), jnp.float32),
        pltpu.VMEM((RACC,), jnp.float32),
        pltpu.VMEM((RACC,), jnp.float32),
        pltpu.VMEM((RACC,), jnp.float32),
        pltpu.VMEM((16,), jnp.int32),
        pltpu.SemaphoreType.DMA,
        pltpu.SemaphoreType.DMA,
    ],
)()


def _assemble(flat):
    return flat.reshape(NPAD, F)[:N_NODES]


def _pre_body(x_ref, wt_ref, wb_ref, pb_ref, a_ref, b_ref):
    x = x_ref[...]
    a_ref[...] = (
        jnp.dot(x, wt_ref[...], preferred_element_type=jnp.float32) + pb_ref[...]
    )
    b_ref[...] = jnp.dot(x, wb_ref[...], preferred_element_type=jnp.float32)


def _pre(x, pre_W, pre_b):
    wt, wb = pre_W[:F], pre_W[F:]
    return pl.pallas_call(
        _pre_body,
        grid=(N_NODES // BLK,),
        in_specs=[
            pl.BlockSpec((BLK, F), lambda i: (i, 0)),
            pl.BlockSpec((F, F), lambda i: (0, 0)),
            pl.BlockSpec((F, F), lambda i: (0, 0)),
            pl.BlockSpec((1, F), lambda i: (0, 0)),
        ],
        out_specs=[
            pl.BlockSpec((BLK, F), lambda i: (i, 0)),
            pl.BlockSpec((BLK, F), lambda i: (i, 0)),
        ],
        out_shape=[
            jax.ShapeDtypeStruct((N_NODES, F), jnp.float32),
            jax.ShapeDtypeStruct((N_NODES, F), jnp.float32),
        ],
    )(x, wt, wb, pre_b.reshape(1, F))


def _post_body(apply_elu, x_ref, a_ref, cnt_ref, s_ref, s2_ref, mn_ref, mx_ref,
               px_ref, p1_ref, p2_ref, p3_ref, pb_ref, lw_ref, lb_ref,
               al_ref, o_ref):
    a = a_ref[...]
    cnt = cnt_ref[...]
    d = jnp.maximum(cnt, 1.0)
    has = cnt > 0.0
    s = cnt * a + s_ref[...]
    mean = s / d
    mean2 = (cnt * a * a + 2.0 * a * s_ref[...] + s2_ref[...]) / d
    var = jnp.maximum(mean2 - mean * mean, 0.0)
    std = jnp.sqrt(var + EPS)
    mn = jnp.where(has, a + mn_ref[...], 0.0)
    mx = jnp.where(has, a + mx_ref[...], 0.0)
    agg = jnp.concatenate([mean, mn, mx, std], axis=-1)
    avg_log = al_ref[0, 0]
    logd = jnp.log(d + 1.0)
    amp = logd / avg_log
    att = avg_log / logd
    t1 = jnp.dot(agg, p1_ref[...], preferred_element_type=jnp.float32)
    t2 = jnp.dot(agg, p2_ref[...], preferred_element_type=jnp.float32)
    t3 = jnp.dot(agg, p3_ref[...], preferred_element_type=jnp.float32)
    out = (
        jnp.dot(x_ref[...], px_ref[...], preferred_element_type=jnp.float32)
        + t1 + amp * t2 + att * t3 + pb_ref[...]
    )
    out = jnp.dot(out, lw_ref[...], preferred_element_type=jnp.float32) + lb_ref[...]
    if apply_elu:
        out = jnp.where(out > 0.0, out, jnp.exp(jnp.minimum(out, 0.0)) - 1.0)
    o_ref[...] = out


def _post(x, a, cnt, S, S2, MN, MX, post_W, post_b, lin_W, lin_b, avg_log,
          apply_elu):
    px, p1, p2, p3 = (post_W[:F], post_W[F:F + 4 * F],
                      post_W[F + 4 * F:F + 8 * F], post_W[F + 8 * F:])
    full = lambda shp: pl.BlockSpec(shp, lambda i: (0, 0))
    blk = pl.BlockSpec((BLK, F), lambda i: (i, 0))
    return pl.pallas_call(
        functools.partial(_post_body, apply_elu),
        grid=(N_NODES // BLK,),
        in_specs=[
            blk, blk,
            pl.BlockSpec((BLK, 1), lambda i: (i, 0)),
            blk, blk, blk, blk,
            full((F, F)), full((4 * F, F)), full((4 * F, F)), full((4 * F, F)),
            full((1, F)), full((F, F)), full((1, F)), full((1, 1)),
        ],
        out_specs=blk,
        out_shape=jax.ShapeDtypeStruct((N_NODES, F), jnp.float32),
    )(x, a, cnt.reshape(-1, 1), S, S2, MN, MX, px, p1, p2, p3,
      post_b.reshape(1, F), lin_W, lin_b.reshape(1, F), avg_log.reshape(1, 1))


def kernel(x, edge_index, deg_hist, pre_W1, pre_b1, post_W1, post_b1, lin_W1,
           lin_b1, pre_W2, pre_b2, post_W2, post_b2, lin_W2, lin_b2):
    src, dst = edge_index[0], edge_index[1]
    bins = jnp.arange(deg_hist.shape[0], dtype=jnp.float32)
    hist = deg_hist.astype(jnp.float32)
    avg_log = jnp.sum(jnp.log(bins + 1.0) * hist) / jnp.sum(hist)

    osrc, odst, cnts, deg = _partition(src, dst)
    cnt = deg.reshape(NR, NPR + 1, 16)[:, :NPR, 0].reshape(NPAD)[:N_NODES]

    a1, b1 = _pre(x, pre_W1, pre_b1)
    sS, sS2, sMN, sMX = _segment(b1, osrc, odst, cnts)
    h = _post(x, a1, cnt, _assemble(sS), _assemble(sS2), _assemble(sMN),
              _assemble(sMX), post_W1, post_b1, lin_W1, lin_b1, avg_log, True)

    a2, b2 = _pre(h, pre_W2, pre_b2)
    tS, tS2, tMN, tMX = _segment(b2, osrc, odst, cnts)
    return _post(h, a2, cnt, _assemble(tS), _assemble(tS2), _assemble(tMN),
                 _assemble(tMX), post_W2, post_b2, lin_W2, lin_b2, avg_log,
                 False)


# speedup vs baseline: 5.5681x; 1.0049x over previous
import functools

import jax
import jax.numpy as jnp
from jax import lax
from jax.experimental import pallas as pl
from jax.experimental.pallas import tpu as pltpu
from jax.experimental.pallas import tpu_sc as plsc

N_NODES = 10000
E_EDGES = 320000
F = 128
EPS = 1e-5
BLK = 400

NC = 2
NS = 16
NW = NC * NS
NR = 64
NPR = 157
NPAD = NR * NPR
CAP = 8192
LISTSZ = CAP + 144
CHUNK = 16000
NCHUNK = E_EDGES // CHUNK
BATCH = 128
SUB = BATCH // 16
RACC = (NPR + 1) * F
ROUT = NPR * F
RCNT = (NPR + 1) * 16
BIG = 3.0e38

_MESH = plsc.VectorSubcoreMesh(core_axis_name="c", subcore_axis_name="s")
_SC_PARAMS = pltpu.CompilerParams(needs_layout_passes=False)


def _wid():
    return lax.axis_index("s") * NC + lax.axis_index("c")


def _partition_body(src_hbm, dst_hbm, osrc_hbm, odst_hbm, cnts_hbm, deg_hbm,
                    src_v, dst_v, osrc_l0, odst_l0, osrc_l1, odst_l1, cnt_v,
                    acc_cnt):
    wid = _wid()
    base_lo = wid * (2 * NPR)

    def chunk_body(c, carry):
        pltpu.sync_copy(src_hbm.at[pl.ds(c * CHUNK, CHUNK)], src_v)
        pltpu.sync_copy(dst_hbm.at[pl.ds(c * CHUNK, CHUNK)], dst_v)

        def scan_body(i, carry):
            off0, off1 = carry
            sA = src_v[pl.ds(i * 32, 16)]
            dA = dst_v[pl.ds(i * 32, 16)]
            sB = src_v[pl.ds(i * 32 + 16, 16)]
            dB = dst_v[pl.ds(i * 32 + 16, 16)]
            dlA = dA - base_lo
            dlB = dB - base_lo
            m0A = (dlA >= 0) & (dlA < NPR)
            m1A = (dlA >= NPR) & (dlA < 2 * NPR)
            m0B = (dlB >= 0) & (dlB < NPR)
            m1B = (dlB >= NPR) & (dlB < 2 * NPR)
            cs0A = plsc.cumsum(m0A.astype(jnp.int32))
            cs1A = plsc.cumsum(m1A.astype(jnp.int32))
            cs0B = plsc.cumsum(m0B.astype(jnp.int32))
            cs1B = plsc.cumsum(m1B.astype(jnp.int32))
            n0A = cs0A[15]
            n1A = cs1A[15]
            p0A = jnp.minimum(off0, CAP) + cs0A - 1
            p1A = jnp.minimum(off1, CAP) + cs1A - 1
            p0B = jnp.minimum(off0 + n0A, CAP) + cs0B - 1
            p1B = jnp.minimum(off1 + n1A, CAP) + cs1B - 1
            plsc.store_scatter(osrc_l0, [p0A], sA, mask=m0A)
            plsc.store_scatter(odst_l0, [p0A], dlA, mask=m0A)
            plsc.store_scatter(osrc_l1, [p1A], sA, mask=m1A)
            plsc.store_scatter(odst_l1, [p1A], dlA - NPR, mask=m1A)
            plsc.store_scatter(osrc_l0, [p0B], sB, mask=m0B)
            plsc.store_scatter(odst_l0, [p0B], dlB, mask=m0B)
            plsc.store_scatter(osrc_l1, [p1B], sB, mask=m1B)
            plsc.store_scatter(odst_l1, [p1B], dlB - NPR, mask=m1B)
            return (off0 + n0A + cs0B[15], off1 + n1A + cs1B[15])

        return lax.fori_loop(0, CHUNK // 32, scan_body, carry)

    off0, off1 = lax.fori_loop(0, NCHUNK, chunk_body,
                               (jnp.int32(0), jnp.int32(0)))
    zeros = jnp.zeros((16,), jnp.int32)
    trash = jnp.full((16,), NPR, jnp.int32)
    for ol, dl_, off in ((osrc_l0, odst_l0, off0), (osrc_l1, odst_l1, off1)):
        offc = jnp.minimum(off, CAP)
        for p in range(8):
            ol[pl.ds(offc + 16 * p, 16)] = zeros
            dl_[pl.ds(offc + 16 * p, 16)] = trash
    cnt_v[...] = jnp.full((16,), jnp.minimum(off0, CAP), jnp.int32)
    pltpu.sync_copy(cnt_v, cnts_hbm.at[2 * wid])
    cnt_v[...] = jnp.full((16,), jnp.minimum(off1, CAP), jnp.int32)
    pltpu.sync_copy(cnt_v, cnts_hbm.at[2 * wid + 1])
    pltpu.sync_copy(osrc_l0, osrc_hbm.at[2 * wid])
    pltpu.sync_copy(odst_l0, odst_hbm.at[2 * wid])
    pltpu.sync_copy(osrc_l1, osrc_hbm.at[2 * wid + 1])
    pltpu.sync_copy(odst_l1, odst_hbm.at[2 * wid + 1])
    onesf = jnp.ones((16,), jnp.float32)
    zerosf = jnp.zeros((16,), jnp.float32)
    for vr, (dl_, off) in enumerate(((odst_l0, off0), (odst_l1, off1))):
        def czero(i, _):
            acc_cnt[pl.ds(i * 16, 16)] = zerosf
            return 0

        lax.fori_loop(0, RCNT // 16, czero, 0)
        nbc = (jnp.minimum(off, CAP) + 15) // 16

        def cbody(i, _, dl_=dl_):
            o16 = dl_[pl.ds(i * 16, 16)]
            for e in range(16):
                d = o16[e]
                plsc.addupdate(acc_cnt.at[pl.ds(d * 16, 16)], onesf)
            return 0

        lax.fori_loop(0, nbc, cbody, 0)
        pltpu.sync_copy(acc_cnt, deg_hbm.at[2 * wid + vr])


_partition = functools.partial(
    pl.kernel, _partition_body, mesh=_MESH,
    compiler_params=_SC_PARAMS,
    out_type=[
        jax.ShapeDtypeStruct((NR, LISTSZ), jnp.int32),
        jax.ShapeDtypeStruct((NR, LISTSZ), jnp.int32),
        jax.ShapeDtypeStruct((NR, 16), jnp.int32),
        jax.ShapeDtypeStruct((NR, RCNT), jnp.float32),
    ],
    scratch_types=[
        pltpu.VMEM((CHUNK,), jnp.int32),
        pltpu.VMEM((CHUNK,), jnp.int32),
        pltpu.VMEM((LISTSZ,), jnp.int32),
        pltpu.VMEM((LISTSZ,), jnp.int32),
        pltpu.VMEM((LISTSZ,), jnp.int32),
        pltpu.VMEM((LISTSZ,), jnp.int32),
        pltpu.VMEM((16,), jnp.int32),
        pltpu.VMEM((RCNT,), jnp.float32),
    ],
)()


def _segment_body(b, osrc_hbm, odst_hbm, cnts_hbm,
                  s_out, s2_out, mn_out, mx_out,
                  osrc_v, odst_v, stag0, stag1, acc_s, acc_s2, acc_mn, acc_mx,
                  cnt_v, sem0, sem1):
    wid = _wid()
    zf = jnp.zeros((16,), jnp.float32)
    big = jnp.full((16,), BIG, jnp.float32)

    for vr in (0, 1):
        vw = 2 * wid + vr
        pltpu.sync_copy(osrc_hbm.at[vw], osrc_v)
        pltpu.sync_copy(odst_hbm.at[vw], odst_v)
        pltpu.sync_copy(cnts_hbm.at[vw], cnt_v)
        cw = cnt_v[...][0]
        nb = (cw + BATCH - 1) // BATCH

        def init_body(i, _):
            acc_s[pl.ds(i * 16, 16)] = zf
            acc_s2[pl.ds(i * 16, 16)] = zf
            acc_mn[pl.ds(i * 16, 16)] = big
            acc_mx[pl.ds(i * 16, 16)] = -big
            return 0

        lax.fori_loop(0, RACC // 16, init_body, 0)

        def issue(j, stagx, semx):
            jb = pl.multiple_of(j * BATCH, BATCH)
            pltpu.async_copy(b.at[osrc_v.at[pl.ds(jb, BATCH)]], stagx, semx)

        def waitb(stagx, semx):
            pltpu.make_async_copy(
                b.at[osrc_v.at[pl.ds(0, BATCH)]], stagx, semx).wait()

        def accum(j, stagx):
            def sub_body(s, _):
                o16 = odst_v[pl.ds(j * BATCH + s * 16, 16)]
                for e in range(16):
                    dl = o16[e]
                    rb = dl * F
                    r = s * 16 + e
                    for k in range(F // 16):
                        v = stagx[r, pl.ds(k * 16, 16)]
                        plsc.addupdate(acc_s.at[pl.ds(rb + k * 16, 16)], v)
                        plsc.addupdate(acc_s2.at[pl.ds(rb + k * 16, 16)], v * v)
                        cm = acc_mn[pl.ds(rb + k * 16, 16)]
                        acc_mn[pl.ds(rb + k * 16, 16)] = jnp.minimum(cm, v)
                        cx = acc_mx[pl.ds(rb + k * 16, 16)]
                        acc_mx[pl.ds(rb + k * 16, 16)] = jnp.maximum(cx, v)
                return 0

            lax.fori_loop(0, SUB, sub_body, 0)

        @pl.when(nb > 0)
        def _():
            issue(0, stag0, sem0)

        def pair_body(p, _):
            j0 = 2 * p
            j1 = j0 + 1

            @pl.when(j1 < nb)
            def _():
                issue(j1, stag1, sem1)

            waitb(stag0, sem0)
            accum(j0, stag0)

            @pl.when(j1 < nb)
            def _():
                @pl.when(j1 + 1 < nb)
                def _():
                    issue(j1 + 1, stag0, sem0)

                waitb(stag1, sem1)
                accum(j1, stag1)

            return 0

        lax.fori_loop(0, (nb + 1) // 2, pair_body, 0)
        base = pl.multiple_of(vw * ROUT, 128)
        pltpu.sync_copy(acc_s.at[pl.ds(0, ROUT)], s_out.at[pl.ds(base, ROUT)])
        pltpu.sync_copy(acc_s2.at[pl.ds(0, ROUT)], s2_out.at[pl.ds(base, ROUT)])
        pltpu.sync_copy(acc_mn.at[pl.ds(0, ROUT)], mn_out.at[pl.ds(base, ROUT)])
        pltpu.sync_copy(acc_mx.at[pl.ds(0, ROUT)], mx_out.at[pl.ds(base, ROUT)])


_segment = functools.partial(
    pl.kernel, _segment_body,
    mesh=_MESH, compiler_params=_SC_PARAMS,
    out_type=[jax.ShapeDtypeStruct((NR * ROUT,), jnp.float32)] * 4,
    scratch_types=[
        pltpu.VMEM((LISTSZ,), jnp.int32),
        pltpu.VMEM((LISTSZ,), jnp.int32),
        pltpu.VMEM((BATCH, F), jnp.float32),
        pltpu.VMEM((BATCH, F), jnp.float32),
        pltpu.VMEM((RACC,), jnp.float32),
        pltpu.VMEM((RACC,), jnp.float32),
        pltpu.VMEM((RACC,), jnp.float32),
        pltpu.VMEM((RACC,), jnp.float32),
        pltpu.VMEM((16,), jnp.int32),
        pltpu.SemaphoreType.DMA,
        pltpu.SemaphoreType.DMA,
    ],
)()


def _assemble(flat):
    return flat.reshape(NPAD, F)[:N_NODES]


def _pre_body(x_ref, wt_ref, wb_ref, pb_ref, a_ref, b_ref):
    x = x_ref[...]
    a_ref[...] = (
        jnp.dot(x, wt_ref[...], preferred_element_type=jnp.float32) + pb_ref[...]
    )
    b_ref[...] = jnp.dot(x, wb_ref[...], preferred_element_type=jnp.float32)


def _pre(x, pre_W, pre_b):
    wt, wb = pre_W[:F], pre_W[F:]
    return pl.pallas_call(
        _pre_body,
        grid=(N_NODES // BLK,),
        in_specs=[
            pl.BlockSpec((BLK, F), lambda i: (i, 0)),
            pl.BlockSpec((F, F), lambda i: (0, 0)),
            pl.BlockSpec((F, F), lambda i: (0, 0)),
            pl.BlockSpec((1, F), lambda i: (0, 0)),
        ],
        out_specs=[
            pl.BlockSpec((BLK, F), lambda i: (i, 0)),
            pl.BlockSpec((BLK, F), lambda i: (i, 0)),
        ],
        out_shape=[
            jax.ShapeDtypeStruct((N_NODES, F), jnp.float32),
            jax.ShapeDtypeStruct((N_NODES, F), jnp.float32),
        ],
    )(x, wt, wb, pre_b.reshape(1, F))


def _post_body(apply_elu, x_ref, a_ref, cnt_ref, s_ref, s2_ref, mn_ref, mx_ref,
               px_ref, p1_ref, p2_ref, p3_ref, pb_ref, lw_ref, lb_ref,
               al_ref, o_ref):
    a = a_ref[...]
    cnt = cnt_ref[...]
    d = jnp.maximum(cnt, 1.0)
    has = cnt > 0.0
    s = cnt * a + s_ref[...]
    mean = s / d
    mean2 = (cnt * a * a + 2.0 * a * s_ref[...] + s2_ref[...]) / d
    var = jnp.maximum(mean2 - mean * mean, 0.0)
    std = jnp.sqrt(var + EPS)
    mn = jnp.where(has, a + mn_ref[...], 0.0)
    mx = jnp.where(has, a + mx_ref[...], 0.0)
    agg = jnp.concatenate([mean, mn, mx, std], axis=-1)
    avg_log = al_ref[0, 0]
    logd = jnp.log(d + 1.0)
    amp = logd / avg_log
    att = avg_log / logd
    t1 = jnp.dot(agg, p1_ref[...], preferred_element_type=jnp.float32)
    t2 = jnp.dot(agg, p2_ref[...], preferred_element_type=jnp.float32)
    t3 = jnp.dot(agg, p3_ref[...], preferred_element_type=jnp.float32)
    out = (
        jnp.dot(x_ref[...], px_ref[...], preferred_element_type=jnp.float32)
        + t1 + amp * t2 + att * t3 + pb_ref[...]
    )
    out = jnp.dot(out, lw_ref[...], preferred_element_type=jnp.float32) + lb_ref[...]
    if apply_elu:
        out = jnp.where(out > 0.0, out, jnp.exp(jnp.minimum(out, 0.0)) - 1.0)
    o_ref[...] = out


def _post(x, a, cnt, S, S2, MN, MX, post_W, post_b, lin_W, lin_b, avg_log,
          apply_elu):
    px, p1, p2, p3 = (post_W[:F], post_W[F:F + 4 * F],
                      post_W[F + 4 * F:F + 8 * F], post_W[F + 8 * F:])
    full = lambda shp: pl.BlockSpec(shp, lambda i: (0, 0))
    blk = pl.BlockSpec((BLK, F), lambda i: (i, 0))
    return pl.pallas_call(
        functools.partial(_post_body, apply_elu),
        grid=(N_NODES // BLK,),
        in_specs=[
            blk, blk,
            pl.BlockSpec((BLK, 1), lambda i: (i, 0)),
            blk, blk, blk, blk,
            full((F, F)), full((4 * F, F)), full((4 * F, F)), full((4 * F, F)),
            full((1, F)), full((F, F)), full((1, F)), full((1, 1)),
        ],
        out_specs=blk,
        out_shape=jax.ShapeDtypeStruct((N_NODES, F), jnp.float32),
    )(x, a, cnt.reshape(-1, 1), S, S2, MN, MX, px, p1, p2, p3,
      post_b.reshape(1, F), lin_W, lin_b.reshape(1, F), avg_log.reshape(1, 1))


def kernel(x, edge_index, deg_hist, pre_W1, pre_b1, post_W1, post_b1, lin_W1,
           lin_b1, pre_W2, pre_b2, post_W2, post_b2, lin_W2, lin_b2):
    src, dst = edge_index[0], edge_index[1]
    bins = jnp.arange(deg_hist.shape[0], dtype=jnp.float32)
    hist = deg_hist.astype(jnp.float32)
    avg_log = jnp.sum(jnp.log(bins + 1.0) * hist) / jnp.sum(hist)

    osrc, odst, cnts, deg = _partition(src, dst)
    cnt = deg.reshape(NR, NPR + 1, 16)[:, :NPR, 0].reshape(NPAD)[:N_NODES]

    a1, b1 = _pre(x, pre_W1, pre_b1)
    sS, sS2, sMN, sMX = _segment(b1, osrc, odst, cnts)
    h = _post(x, a1, cnt, _assemble(sS), _assemble(sS2), _assemble(sMN),
              _assemble(sMX), post_W1, post_b1, lin_W1, lin_b1, avg_log, True)

    a2, b2 = _pre(h, pre_W2, pre_b2)
    tS, tS2, tMN, tMX = _segment(b2, osrc, odst, cnts)
    return _post(h, a2, cnt, _assemble(tS), _assemble(tS2), _assemble(tMN),
                 _assemble(tMX), post_W2, post_b2, lin_W2, lin_b2, avg_log,
                 False)
